# baseline (speedup 1.0000x reference)
import functools

import jax
import jax.numpy as jnp
from jax import lax
from jax.experimental import pallas as pl
from jax.experimental.pallas import tpu as pltpu
from jax.experimental.pallas import tpu_sc as plsc

SCALE = 8.0


def kernel(X, table):
    R, S = X.shape
    V, D = table.shape
    B = R * S

    info = plsc.get_sparse_core_info()
    NC, NS = info.num_cores, info.num_subcores
    NW = NC * NS
    CH = 128
    n_ch = B // (NW * CH)
    assert B == NW * n_ch * CH

    idx = X.reshape(NW, n_ch, CH).astype(jnp.int32)

    mesh = plsc.VectorSubcoreMesh(core_axis_name="c", subcore_axis_name="s")

    @functools.partial(
        pl.kernel,
        mesh=mesh,
        out_type=jax.ShapeDtypeStruct((B, D), jnp.float32),
        compiler_params=pltpu.CompilerParams(use_tc_tiling_on_sc=False),
        scratch_types=[
            pltpu.VMEM((n_ch, CH), jnp.int32),
            pltpu.VMEM((CH, D), jnp.float32),
            pltpu.SemaphoreType.DMA,
        ],
    )
    def sc_kernel(idx_hbm, table_hbm, out_hbm, idx_v, rows_v, sem):
        wid = lax.axis_index("s") * NC + lax.axis_index("c")
        base = wid * (n_ch * CH)
        pltpu.sync_copy(idx_hbm.at[wid], idx_v)

        def chunk_body(j, carry):
            pltpu.async_copy(table_hbm.at[idx_v.at[j]], rows_v, sem).wait()

            def scale_body(r, c2):
                for c in range(D // 16):
                    sl = pl.ds(c * 16, 16)
                    rows_v[r, sl] = rows_v[r, sl] * SCALE
                return c2

            lax.fori_loop(0, CH, scale_body, 0)
            pltpu.sync_copy(rows_v, out_hbm.at[pl.ds(base + j * CH, CH)])
            return carry

        lax.fori_loop(0, n_ch, chunk_body, 0)

    out = sc_kernel(idx, table)
    return out.reshape(R, S, D)


# speedup vs baseline: 1.2014x; 1.2014x over previous
import functools

import jax
import jax.numpy as jnp
from jax import lax
from jax.experimental import pallas as pl
from jax.experimental.pallas import tpu as pltpu
from jax.experimental.pallas import tpu_sc as plsc

SCALE = 8.0


def kernel(X, table):
    R, S = X.shape
    V, D = table.shape
    B = R * S

    info = plsc.get_sparse_core_info()
    NC, NS = info.num_cores, info.num_subcores
    NW = NC * NS
    CH = 128
    n_ch = B // (NW * CH)
    assert B == NW * n_ch * CH
    NBUF = 4
    A = 2
    assert n_ch % NBUF == 0

    idx = X.reshape(NW, n_ch, CH).astype(jnp.int32)

    mesh = plsc.VectorSubcoreMesh(core_axis_name="c", subcore_axis_name="s")

    @functools.partial(
        pl.kernel,
        mesh=mesh,
        out_type=jax.ShapeDtypeStruct((B, D), jnp.float32),
        compiler_params=pltpu.CompilerParams(use_tc_tiling_on_sc=False),
        scratch_types=[
            pltpu.VMEM((n_ch, CH), jnp.int32),
            pltpu.VMEM((NBUF, CH, D), jnp.float32),
            pltpu.SemaphoreType.DMA((NBUF,)),
            pltpu.SemaphoreType.DMA((NBUF,)),
        ],
    )
    def sc_kernel(idx_hbm, table_hbm, out_hbm, idx_v, rows_v, gsem, ssem):
        wid = lax.axis_index("s") * NC + lax.axis_index("c")
        base = wid * (n_ch * CH)
        pltpu.sync_copy(idx_hbm.at[wid], idx_v)

        for c in range(A):
            pltpu.async_copy(
                table_hbm.at[idx_v.at[c]], rows_v.at[c], gsem.at[c]
            )

        @pl.loop(0, n_ch, step=NBUF)
        def outer(j):
            for b in range(NBUF):
                jj = j + b
                nxt = jj + A
                b2 = (b + A) % NBUF

                @pl.when(jnp.logical_and(nxt < n_ch, nxt >= NBUF))
                def _fire_steady():
                    pltpu.make_async_copy(
                        rows_v.at[b2],
                        out_hbm.at[pl.ds(base, CH)],
                        ssem.at[b2],
                    ).wait()
                    pltpu.async_copy(
                        table_hbm.at[idx_v.at[nxt]], rows_v.at[b2], gsem.at[b2]
                    )

                @pl.when(jnp.logical_and(nxt < n_ch, nxt < NBUF))
                def _fire_first():
                    pltpu.async_copy(
                        table_hbm.at[idx_v.at[nxt]], rows_v.at[b2], gsem.at[b2]
                    )

                pltpu.make_async_copy(
                    table_hbm.at[idx_v.at[b]], rows_v.at[b], gsem.at[b]
                ).wait()

                @pl.loop(0, CH)
                def _scale(r):
                    for c in range(D // 16):
                        sl = pl.ds(c * 16, 16)
                        rows_v[b, r, sl] = rows_v[b, r, sl] * SCALE

                pltpu.async_copy(
                    rows_v.at[b],
                    out_hbm.at[pl.ds(base + jj * CH, CH)],
                    ssem.at[b],
                )

        for b in range(NBUF):
            pltpu.make_async_copy(
                rows_v.at[b], out_hbm.at[pl.ds(base, CH)], ssem.at[b]
            ).wait()

    out = sc_kernel(idx, table)
    return out.reshape(R, S, D)
